# Initial kernel scaffold; baseline (speedup 1.0000x reference)
import jax
import jax.numpy as jnp
from jax.experimental import pallas as pl


def kernel(h, lid, rid, labels):
    raise NotImplementedError("write your pallas kernel here")


# speedup vs baseline: 1.0655x; 1.0655x over previous
import functools

import jax
import jax.numpy as jnp
from jax import lax
from jax.experimental import pallas as pl
from jax.experimental.pallas import tpu as pltpu
from jax.experimental.pallas import tpu_sc as plsc

_MARGIN = 10.0

_V = 10000
_D = 128
_E = 320000
_NW = 32
_PER_W = _E // _NW
_CHUNK = 80
_NCHUNK = _PER_W // _CHUNK
_NGROUP = _CHUNK // 16


def _sqrt16(x):
    xc = jnp.maximum(x, jnp.float32(1e-30))
    i = plsc.bitcast(xc, jnp.int32)
    y = plsc.bitcast(jnp.int32(0x5F3759DF) - (i >> 1), jnp.float32)
    xh = xc * jnp.float32(0.5)
    for _ in range(3):
        y = y * (jnp.float32(1.5) - xh * y * y)
    return xc * y


def _sc_body(lid_hbm, rid_hbm, lab_hbm, h_hbm, out_hbm,
             idx_l, idx_r, labv, lrows, rrows, outv, sem):
    nc = plsc.get_sparse_core_info().num_cores
    wid = lax.axis_index("s") * nc + lax.axis_index("c")
    base = wid * _PER_W
    lane = lax.iota(jnp.int32, 16)

    def chunk_body(c, carry):
        acc_a, acc_b, acc_c = carry
        off = base + c * _CHUNK
        pltpu.sync_copy(lid_hbm.at[pl.ds(off, _CHUNK)], idx_l)
        pltpu.sync_copy(rid_hbm.at[pl.ds(off, _CHUNK)], idx_r)
        pltpu.sync_copy(lab_hbm.at[pl.ds(off, _CHUNK)], labv)
        cp_l = pltpu.async_copy(h_hbm.at[idx_l], lrows, sem)
        cp_r = pltpu.async_copy(h_hbm.at[idx_r], rrows, sem)
        cp_l.wait()
        cp_r.wait()

        for g in range(_NGROUP):
            pids = lane + jnp.int32(g * 16)

            def d_body(d, gcarry):
                g_d, g_b, g_c = gcarry
                col = jnp.full((16,), d, jnp.int32)
                lv = plsc.load_gather(lrows, [pids, col])
                rv = plsc.load_gather(rrows, [pids, col])
                diff = lv - rv
                return (g_d + diff * diff, g_b + lv * lv, g_c + rv * rv)

            zero = jnp.zeros((16,), jnp.float32)
            g_d, g_b, g_c = lax.fori_loop(0, _D, d_body, (zero, zero, zero))
            lab = labv[pl.ds(g * 16, 16)]
            w = jnp.where(lab == 0, jnp.float32(-1.0), jnp.float32(1.0))
            acc_a = acc_a + w * _sqrt16(g_d)
            acc_b = acc_b + g_b
            acc_c = acc_c + g_c
        return (acc_a, acc_b, acc_c)

    zero = jnp.zeros((16,), jnp.float32)
    acc_a, acc_b, acc_c = lax.fori_loop(
        0, _NCHUNK, chunk_body, (zero, zero, zero))
    outv[pl.ds(0, 16)] = acc_a
    outv[pl.ds(16, 16)] = acc_b
    outv[pl.ds(32, 16)] = acc_c
    pltpu.sync_copy(outv, out_hbm.at[wid])


_sc_partials = functools.partial(
    pl.kernel,
    out_type=jax.ShapeDtypeStruct((_NW, 48), jnp.float32),
    mesh=plsc.VectorSubcoreMesh(core_axis_name="c", subcore_axis_name="s"),
    compiler_params=pltpu.CompilerParams(needs_layout_passes=False),
    scratch_types=[
        pltpu.VMEM((_CHUNK,), jnp.int32),
        pltpu.VMEM((_CHUNK,), jnp.int32),
        pltpu.VMEM((_CHUNK,), jnp.int32),
        pltpu.VMEM((_CHUNK, _D), jnp.float32),
        pltpu.VMEM((_CHUNK, _D), jnp.float32),
        pltpu.VMEM((48,), jnp.float32),
        pltpu.SemaphoreType.DMA,
    ],
)(_sc_body)


def _combine_body(p_ref, o_ref):
    x = p_ref[...]
    a = jnp.sum(x[:, 0:16])
    b = jnp.sum(x[:, 16:32])
    c = jnp.sum(x[:, 32:48])
    denom = jnp.sqrt(jnp.sqrt(b) * jnp.sqrt(c))
    o_ref[...] = jnp.reshape(
        jnp.maximum(a / denom + jnp.float32(_MARGIN), jnp.float32(0.0)),
        (1, 1))


_combine = pl.pallas_call(
    _combine_body,
    out_shape=jax.ShapeDtypeStruct((1, 1), jnp.float32),
)


def kernel(h, lid, rid, labels):
    partials = _sc_partials(lid, rid, labels, h)
    return _combine(partials)[0, 0]


# speedup vs baseline: 1.2768x; 1.1983x over previous
import functools

import jax
import jax.numpy as jnp
from jax import lax
from jax.experimental import pallas as pl
from jax.experimental.pallas import tpu as pltpu
from jax.experimental.pallas import tpu_sc as plsc

_MARGIN = 10.0

_D = 128
_E = 320000
_NW = 32
_PER_W = 10240
_EPAD = _NW * _PER_W
_CHUNK = 128
_NCH = _PER_W // _CHUNK
_NGRP = _CHUNK // 16


def _sqrt16(x):
    xc = jnp.maximum(x, jnp.float32(1e-30))
    i = plsc.bitcast(xc, jnp.int32)
    y = plsc.bitcast(jnp.int32(0x5F3759DF) - (i >> 1), jnp.float32)
    xh = xc * jnp.float32(0.5)
    for _ in range(3):
        y = y * (jnp.float32(1.5) - xh * y * y)
    return xc * y


def _sc_body(lid_hbm, rid_hbm, lab_hbm, h_hbm, out_hbm,
             lid_v, rid_v, lab_v, lrows0, rrows0, lrows1, rrows1, outv,
             sem0, sem1):
    nc = plsc.get_sparse_core_info().num_cores
    wid = lax.axis_index("s") * nc + lax.axis_index("c")
    base = wid * _PER_W
    pltpu.sync_copy(lid_hbm.at[pl.ds(base, _PER_W)], lid_v)
    pltpu.sync_copy(rid_hbm.at[pl.ds(base, _PER_W)], rid_v)
    pltpu.sync_copy(lab_hbm.at[pl.ds(base, _PER_W)], lab_v)

    lbufs = (lrows0, lrows1)
    rbufs = (rrows0, rrows1)
    sems = (sem0, sem1)
    lane = lax.iota(jnp.int32, 16)

    def issue(c, b):
        idx = pl.ds(c * _CHUNK, _CHUNK)
        pltpu.async_copy(h_hbm.at[lid_v.at[idx]], lbufs[b], sems[b])
        pltpu.async_copy(h_hbm.at[rid_v.at[idx]], rbufs[b], sems[b])

    issue(jnp.int32(0), 0)

    def compute_chunk(c, b, carry):
        acc_a, acc_b, acc_c = carry
        idx = pl.ds(c * _CHUNK, _CHUNK)
        pltpu.make_async_copy(h_hbm.at[lid_v.at[idx]], lbufs[b], sems[b]).wait()
        pltpu.make_async_copy(h_hbm.at[rid_v.at[idx]], rbufs[b], sems[b]).wait()

        @pl.when(c + 1 < _NCH)
        def _():
            issue(c + 1, 1 - b)

        zero = jnp.zeros((16,), jnp.float32)
        for g in range(_NGRP):
            pids = lane + jnp.int32(g * 16)

            @plsc.parallel_loop(0, _D, 8, carry=(zero,) * 6)
            def dloop(d, gc, _pids=pids, _lb=lbufs[b], _rb=rbufs[b]):
                d0, d1, b0, b1, c0, c1 = gc
                for u in range(8):
                    col = jnp.full((16,), d + u, jnp.int32)
                    lv = plsc.load_gather(_lb, [_pids, col])
                    rv = plsc.load_gather(_rb, [_pids, col])
                    diff = lv - rv
                    if u % 2 == 0:
                        d0 = d0 + diff * diff
                        b0 = b0 + lv * lv
                        c0 = c0 + rv * rv
                    else:
                        d1 = d1 + diff * diff
                        b1 = b1 + lv * lv
                        c1 = c1 + rv * rv
                return (d0, d1, b0, b1, c0, c1)

            d0, d1, b0, b1, c0, c1 = dloop
            lab = lab_v[pl.ds(c * _CHUNK + g * 16, 16)]
            w = jnp.where(lab < 0, jnp.float32(0.0),
                          jnp.where(lab == 0, jnp.float32(-1.0),
                                    jnp.float32(1.0)))
            m = jnp.where(lab < 0, jnp.float32(0.0), jnp.float32(1.0))
            acc_a = acc_a + w * _sqrt16(d0 + d1)
            acc_b = acc_b + m * (b0 + b1)
            acc_c = acc_c + m * (c0 + c1)
        return (acc_a, acc_b, acc_c)

    def outer(cc, carry):
        carry = compute_chunk(cc * 2, 0, carry)
        carry = compute_chunk(cc * 2 + 1, 1, carry)
        return carry

    zero = jnp.zeros((16,), jnp.float32)
    acc_a, acc_b, acc_c = lax.fori_loop(
        0, _NCH // 2, outer, (zero, zero, zero))
    outv[pl.ds(0, 16)] = acc_a
    outv[pl.ds(16, 16)] = acc_b
    outv[pl.ds(32, 16)] = acc_c
    pltpu.sync_copy(outv, out_hbm.at[wid])


_sc_partials = functools.partial(
    pl.kernel,
    out_type=jax.ShapeDtypeStruct((_NW, 48), jnp.float32),
    mesh=plsc.VectorSubcoreMesh(core_axis_name="c", subcore_axis_name="s"),
    compiler_params=pltpu.CompilerParams(needs_layout_passes=False),
    scratch_types=[
        pltpu.VMEM((_PER_W,), jnp.int32),
        pltpu.VMEM((_PER_W,), jnp.int32),
        pltpu.VMEM((_PER_W,), jnp.int32),
        pltpu.VMEM((_CHUNK, _D), jnp.float32),
        pltpu.VMEM((_CHUNK, _D), jnp.float32),
        pltpu.VMEM((_CHUNK, _D), jnp.float32),
        pltpu.VMEM((_CHUNK, _D), jnp.float32),
        pltpu.VMEM((48,), jnp.float32),
        pltpu.SemaphoreType.DMA,
        pltpu.SemaphoreType.DMA,
    ],
)(_sc_body)


def _combine_body(p_ref, o_ref):
    x = p_ref[...]
    a = jnp.sum(x[:, 0:16])
    b = jnp.sum(x[:, 16:32])
    c = jnp.sum(x[:, 32:48])
    denom = jnp.sqrt(jnp.sqrt(b) * jnp.sqrt(c))
    o_ref[...] = jnp.reshape(
        jnp.maximum(a / denom + jnp.float32(_MARGIN), jnp.float32(0.0)),
        (1, 1))


_combine = pl.pallas_call(
    _combine_body,
    out_shape=jax.ShapeDtypeStruct((1, 1), jnp.float32),
)


def kernel(h, lid, rid, labels):
    npad = _EPAD - _E
    lid_p = jnp.concatenate([lid, jnp.zeros((npad,), lid.dtype)])
    rid_p = jnp.concatenate([rid, jnp.zeros((npad,), rid.dtype)])
    lab_p = jnp.concatenate([labels, jnp.full((npad,), -1, labels.dtype)])
    partials = _sc_partials(lid_p, rid_p, lab_p, h)
    return _combine(partials)[0, 0]


# speedup vs baseline: 1.5428x; 1.2084x over previous
import functools

import jax
import jax.numpy as jnp
from jax import lax
from jax.experimental import pallas as pl
from jax.experimental.pallas import tpu as pltpu
from jax.experimental.pallas import tpu_sc as plsc

_MARGIN = 10.0

_V = 10000
_D = 128
_E = 320000
_NW = 32
_PER_W = 10240
_EPAD = _NW * _PER_W
_CHUNK = 128
_NCH = _PER_W // _CHUNK
_NGRP = _CHUNK // 16


def _sqrt16(x):
    xc = jnp.maximum(x, jnp.float32(1e-30))
    i = plsc.bitcast(xc, jnp.int32)
    y = plsc.bitcast(jnp.int32(0x5F3759DF) - (i >> 1), jnp.float32)
    xh = xc * jnp.float32(0.5)
    for _ in range(3):
        y = y * (jnp.float32(1.5) - xh * y * y)
    return xc * y


def _sc_body(lid_hbm, rid_hbm, lab_hbm, h_hbm, sq_hbm, out_hbm,
             lid_v, rid_v, lab_v, lrows0, rrows0, lrows1, rrows1,
             sql0, sqr0, sql1, sqr1, outv, sem0, sem1):
    nc = plsc.get_sparse_core_info().num_cores
    wid = lax.axis_index("s") * nc + lax.axis_index("c")
    base = wid * _PER_W
    pltpu.sync_copy(lid_hbm.at[pl.ds(base, _PER_W)], lid_v)
    pltpu.sync_copy(rid_hbm.at[pl.ds(base, _PER_W)], rid_v)
    pltpu.sync_copy(lab_hbm.at[pl.ds(base, _PER_W)], lab_v)

    lbufs = (lrows0, lrows1)
    rbufs = (rrows0, rrows1)
    sqlb = (sql0, sql1)
    sqrb = (sqr0, sqr1)
    sems = (sem0, sem1)
    lane = lax.iota(jnp.int32, 16)

    def issue(c, b):
        idx = pl.ds(c * _CHUNK, _CHUNK)
        pltpu.async_copy(h_hbm.at[lid_v.at[idx]], lbufs[b], sems[b])
        pltpu.async_copy(h_hbm.at[rid_v.at[idx]], rbufs[b], sems[b])
        pltpu.async_copy(sq_hbm.at[lid_v.at[idx]], sqlb[b], sems[b])
        pltpu.async_copy(sq_hbm.at[rid_v.at[idx]], sqrb[b], sems[b])

    issue(jnp.int32(0), 0)

    def compute_chunk(c, b, carry):
        acc_a, acc_b, acc_c = carry
        idx = pl.ds(c * _CHUNK, _CHUNK)
        pltpu.make_async_copy(h_hbm.at[lid_v.at[idx]], lbufs[b], sems[b]).wait()
        pltpu.make_async_copy(h_hbm.at[rid_v.at[idx]], rbufs[b], sems[b]).wait()
        pltpu.make_async_copy(sq_hbm.at[lid_v.at[idx]], sqlb[b], sems[b]).wait()
        pltpu.make_async_copy(sq_hbm.at[rid_v.at[idx]], sqrb[b], sems[b]).wait()

        @pl.when(c + 1 < _NCH)
        def _():
            issue(c + 1, 1 - b)

        zero = jnp.zeros((16,), jnp.float32)
        for g in range(_NGRP):
            pids = lane + jnp.int32(g * 16)

            @plsc.parallel_loop(0, _D, 8, carry=(zero,) * 4)
            def dloop(d, gc, _pids=pids, _lb=lbufs[b], _rb=rbufs[b]):
                a0, a1, a2, a3 = gc
                colbase = lane + jnp.full((16,), d, jnp.int32)
                acc = [a0, a1, a2, a3]
                for u in range(8):
                    col = (colbase + jnp.int32(u)) & jnp.int32(_D - 1)
                    lv = plsc.load_gather(_lb, [_pids, col])
                    rv = plsc.load_gather(_rb, [_pids, col])
                    diff = lv - rv
                    acc[u % 4] = acc[u % 4] + diff * diff
                return tuple(acc)

            a0, a1, a2, a3 = dloop
            g_d = (a0 + a1) + (a2 + a3)
            sql = sqlb[b][pl.ds(g * 16, 16)]
            sqr = sqrb[b][pl.ds(g * 16, 16)]
            lab = lab_v[pl.ds(c * _CHUNK + g * 16, 16)]
            w = jnp.where(lab < 0, jnp.float32(0.0),
                          jnp.where(lab == 0, jnp.float32(-1.0),
                                    jnp.float32(1.0)))
            m = jnp.where(lab < 0, jnp.float32(0.0), jnp.float32(1.0))
            acc_a = acc_a + w * _sqrt16(g_d)
            acc_b = acc_b + m * sql
            acc_c = acc_c + m * sqr
        return (acc_a, acc_b, acc_c)

    def outer(cc, carry):
        carry = compute_chunk(cc * 2, 0, carry)
        carry = compute_chunk(cc * 2 + 1, 1, carry)
        return carry

    zero = jnp.zeros((16,), jnp.float32)
    acc_a, acc_b, acc_c = lax.fori_loop(
        0, _NCH // 2, outer, (zero, zero, zero))
    outv[pl.ds(0, 16)] = acc_a
    outv[pl.ds(16, 16)] = acc_b
    outv[pl.ds(32, 16)] = acc_c
    pltpu.sync_copy(outv, out_hbm.at[wid])


_sc_partials = functools.partial(
    pl.kernel,
    out_type=jax.ShapeDtypeStruct((_NW, 48), jnp.float32),
    mesh=plsc.VectorSubcoreMesh(core_axis_name="c", subcore_axis_name="s"),
    compiler_params=pltpu.CompilerParams(needs_layout_passes=False),
    scratch_types=[
        pltpu.VMEM((_PER_W,), jnp.int32),
        pltpu.VMEM((_PER_W,), jnp.int32),
        pltpu.VMEM((_PER_W,), jnp.int32),
        pltpu.VMEM((_CHUNK, _D), jnp.float32),
        pltpu.VMEM((_CHUNK, _D), jnp.float32),
        pltpu.VMEM((_CHUNK, _D), jnp.float32),
        pltpu.VMEM((_CHUNK, _D), jnp.float32),
        pltpu.VMEM((_CHUNK,), jnp.float32),
        pltpu.VMEM((_CHUNK,), jnp.float32),
        pltpu.VMEM((_CHUNK,), jnp.float32),
        pltpu.VMEM((_CHUNK,), jnp.float32),
        pltpu.VMEM((48,), jnp.float32),
        pltpu.SemaphoreType.DMA,
        pltpu.SemaphoreType.DMA,
    ],
)(_sc_body)


def _sq_body(h_ref, o_ref):
    x = h_ref[...]
    o_ref[...] = jnp.sum(x * x, axis=1)


_sq_rows = pl.pallas_call(
    _sq_body,
    out_shape=jax.ShapeDtypeStruct((_V,), jnp.float32),
)


def _combine_body(p_ref, o_ref):
    x = p_ref[...]
    a = jnp.sum(x[:, 0:16])
    b = jnp.sum(x[:, 16:32])
    c = jnp.sum(x[:, 32:48])
    denom = jnp.sqrt(jnp.sqrt(b) * jnp.sqrt(c))
    o_ref[...] = jnp.reshape(
        jnp.maximum(a / denom + jnp.float32(_MARGIN), jnp.float32(0.0)),
        (1, 1))


_combine = pl.pallas_call(
    _combine_body,
    out_shape=jax.ShapeDtypeStruct((1, 1), jnp.float32),
)


def kernel(h, lid, rid, labels):
    npad = _EPAD - _E
    lid_p = jnp.concatenate([lid, jnp.zeros((npad,), lid.dtype)])
    rid_p = jnp.concatenate([rid, jnp.zeros((npad,), rid.dtype)])
    lab_p = jnp.concatenate([labels, jnp.full((npad,), -1, labels.dtype)])
    sq = _sq_rows(h)
    partials = _sc_partials(lid_p, rid_p, lab_p, h, sq)
    return _combine(partials)[0, 0]
